# baseline (speedup 1.0000x reference)
import functools

import jax
import jax.numpy as jnp
from jax.experimental import pallas as pl
from jax.experimental.pallas import tpu as pltpu

V = 100000
D = 1024
LAT = 576
H = 16
HD = D // H
L = 4
ND = 3
E = 8
K = 2
FH = 1024
B = 1
S = 2048

EMB_BT = 16
BS = 256
BQ = 256
LM_BN = 1024

_F32 = jnp.float32
_ARB = lambda n: pltpu.CompilerParams(dimension_semantics=("arbitrary",) * n)


def _ln(x, g, b):
    m = jnp.mean(x, axis=-1, keepdims=True)
    v = jnp.var(x, axis=-1, keepdims=True)
    return (x - m) / jnp.sqrt(v + 1e-5) * g + b


def _rope(x, pos):
    half = x.shape[-1] // 2
    freqs = 1.0 / (10000.0 ** (jnp.arange(half, dtype=jnp.float32) / half))
    ang = pos[:, None].astype(jnp.float32) * freqs[None, :]
    cos = jnp.cos(ang)[None, None, :, :]
    sin = jnp.sin(ang)[None, None, :, :]
    x1 = x[..., :half]
    x2 = x[..., half:]
    return jnp.concatenate([x1 * cos - x2 * sin, x1 * sin + x2 * cos], axis=-1)


def _embed_body(ids_ref, *refs):
    del ids_ref
    out_ref = refs[-1]
    for j in range(EMB_BT):
        out_ref[0, pl.ds(j, 1), :] = refs[j][0]


def _embed(tokens, emb):
    def row_map(i, ids, j=0):
        return (ids[i * EMB_BT + j], 0, 0)

    emb3 = emb.reshape(V, 1, D)
    in_specs = [
        pl.BlockSpec((1, 1, D), functools.partial(row_map, j=j))
        for j in range(EMB_BT)
    ]
    grid_spec = pltpu.PrefetchScalarGridSpec(
        num_scalar_prefetch=1,
        grid=(S // EMB_BT,),
        in_specs=in_specs,
        out_specs=pl.BlockSpec((1, EMB_BT, D), lambda i, ids: (i, 0, 0)),
    )
    out = pl.pallas_call(
        _embed_body,
        grid_spec=grid_spec,
        out_shape=jax.ShapeDtypeStruct((S // EMB_BT, EMB_BT, D), _F32),
    )(tokens, *([emb3] * EMB_BT))
    return out.reshape(S, D)


def _mm_body(a_ref, w_ref, o_ref):
    o_ref[...] = jnp.dot(a_ref[...], w_ref[...], preferred_element_type=_F32)


def _mm(a, w):
    ka, nb = w.shape
    return pl.pallas_call(
        _mm_body,
        grid=(S // BS,),
        in_specs=[pl.BlockSpec((BS, ka), lambda i: (i, 0)),
                  pl.BlockSpec((ka, nb), lambda i: (0, 0))],
        out_specs=pl.BlockSpec((BS, nb), lambda i: (i, 0)),
        out_shape=jax.ShapeDtypeStruct((S, nb), _F32),
        compiler_params=_ARB(1),
    )(a, w)


def _qk_body(q_ref, k_ref, s_ref):
    qb = pl.program_id(1)
    s = jnp.dot(q_ref[0], k_ref[0].T, preferred_element_type=_F32)
    s = s / jnp.sqrt(jnp.float32(HD))
    row = qb * BQ + jax.lax.broadcasted_iota(jnp.int32, (BQ, S), 0)
    col = jax.lax.broadcasted_iota(jnp.int32, (BQ, S), 1)
    s_ref[0] = jnp.where(col <= row, s, jnp.float32(-1e9))


def _qk(q, k):
    return pl.pallas_call(
        _qk_body,
        grid=(H, S // BQ),
        in_specs=[
            pl.BlockSpec((1, BQ, HD), lambda h, i: (h, i, 0)),
            pl.BlockSpec((1, S, HD), lambda h, i: (h, 0, 0)),
        ],
        out_specs=pl.BlockSpec((1, BQ, S), lambda h, i: (h, i, 0)),
        out_shape=jax.ShapeDtypeStruct((H, S, S), _F32),
        compiler_params=pltpu.CompilerParams(
            dimension_semantics=("arbitrary", "arbitrary")),
    )(q, k)


def _ev_body(e_ref, v_ref, o_ref):
    o_ref[0] = jnp.dot(e_ref[0], v_ref[0], preferred_element_type=_F32)


def _ev(e, v):
    return pl.pallas_call(
        _ev_body,
        grid=(H, S // BQ),
        in_specs=[
            pl.BlockSpec((1, BQ, S), lambda h, i: (h, i, 0)),
            pl.BlockSpec((1, S, HD), lambda h, i: (h, 0, 0)),
        ],
        out_specs=pl.BlockSpec((1, BQ, HD), lambda h, i: (h, i, 0)),
        out_shape=jax.ShapeDtypeStruct((H, S, HD), _F32),
        compiler_params=pltpu.CompilerParams(
            dimension_semantics=("arbitrary", "arbitrary")),
    )(e, v)


def _outproj_body(o_ref, wo_ref, x_ref, out_ref):
    out_ref[...] = x_ref[...] + jnp.dot(
        o_ref[...], wo_ref[...], preferred_element_type=_F32)


def _outproj(o, wo, x):
    row = pl.BlockSpec((BS, D), lambda i: (i, 0))
    return pl.pallas_call(
        _outproj_body,
        grid=(S // BS,),
        in_specs=[row, pl.BlockSpec((D, D), lambda i: (0, 0)), row],
        out_specs=row,
        out_shape=jax.ShapeDtypeStruct((S, D), _F32),
        compiler_params=_ARB(1),
    )(o, wo, x)


def _ffn_body(h_ref, x_ref, w1_ref, w2_ref, out_ref):
    f = jnp.maximum(
        jnp.dot(h_ref[...], w1_ref[...], preferred_element_type=_F32), 0.0)
    out_ref[...] = x_ref[...] + jnp.dot(
        f, w2_ref[...], preferred_element_type=_F32)


def _ffn(h2, x, w1, w2):
    row = pl.BlockSpec((BS, D), lambda i: (i, 0))
    return pl.pallas_call(
        _ffn_body,
        grid=(S // BS,),
        in_specs=[row, row,
                  pl.BlockSpec((D, FH), lambda i: (0, 0)),
                  pl.BlockSpec((FH, D), lambda i: (0, 0))],
        out_specs=row,
        out_shape=jax.ShapeDtypeStruct((S, D), _F32),
        compiler_params=_ARB(1),
    )(h2, x, w1, w2)


def _moe_body(h_ref, x_ref, cent_ref, we1_ref, we2_ref,
              out_ref, rl_ref, tk_ref, acc_ref):
    e = pl.program_id(1)
    h2 = h_ref[...]
    rl = jnp.dot(h2, cent_ref[...].T, preferred_element_type=_F32)

    ii = jax.lax.broadcasted_iota(jnp.int32, (BS, E), 1)
    t1v = jnp.max(rl, axis=1, keepdims=True)
    t1i = jnp.min(jnp.where(rl == t1v, ii, E), axis=1, keepdims=True)
    masked = jnp.where(ii == t1i, -jnp.inf, rl)
    t2v = jnp.max(masked, axis=1, keepdims=True)
    t2i = jnp.min(jnp.where(masked == t2v, ii, E), axis=1, keepdims=True)

    w2 = jnp.exp(t2v - t1v)
    den = 1.0 + w2
    g1 = 1.0 / den
    g2 = w2 / den
    gate = (jnp.where(t1i == e, g1, 0.0) + jnp.where(t2i == e, g2, 0.0))

    he = jnp.maximum(
        jnp.dot(h2, we1_ref[0], preferred_element_type=_F32), 0.0)
    ye = jnp.dot(he, we2_ref[0], preferred_element_type=_F32)
    contrib = ye * gate

    @pl.when(e == 0)
    def _():
        acc_ref[...] = contrib
        rl_ref[...] = rl
        ki = jax.lax.broadcasted_iota(jnp.int32, (BS, K), 1)
        tk_ref[...] = jnp.where(ki == 0, t1i, t2i)

    @pl.when(e > 0)
    def _():
        acc_ref[...] = acc_ref[...] + contrib

    @pl.when(e == E - 1)
    def _():
        out_ref[...] = x_ref[...] + acc_ref[...]


def _moe(h2, x, cent, we1, we2):
    row = pl.BlockSpec((BS, D), lambda i, e: (i, 0))
    return pl.pallas_call(
        _moe_body,
        grid=(S // BS, E),
        in_specs=[
            row,
            row,
            pl.BlockSpec((E, D), lambda i, e: (0, 0)),
            pl.BlockSpec((1, D, FH), lambda i, e: (e, 0, 0)),
            pl.BlockSpec((1, FH, D), lambda i, e: (e, 0, 0)),
        ],
        out_specs=[
            row,
            pl.BlockSpec((BS, E), lambda i, e: (i, 0)),
            pl.BlockSpec((BS, K), lambda i, e: (i, 0)),
        ],
        out_shape=[
            jax.ShapeDtypeStruct((S, D), _F32),
            jax.ShapeDtypeStruct((S, E), _F32),
            jax.ShapeDtypeStruct((S, K), jnp.int32),
        ],
        scratch_shapes=[pltpu.VMEM((BS, D), _F32)],
        compiler_params=pltpu.CompilerParams(
            dimension_semantics=("arbitrary", "arbitrary")),
    )(h2, x, cent, we1, we2)


def _lm_body(x_ref, w_ref, b_ref, out_ref):
    out_ref[...] = jnp.dot(
        x_ref[...], w_ref[...], preferred_element_type=_F32) + b_ref[...]


def _lm_head(x, w, b):
    nb = pl.cdiv(V, LM_BN)
    return pl.pallas_call(
        _lm_body,
        grid=(nb,),
        in_specs=[
            pl.BlockSpec((S, D), lambda n: (0, 0)),
            pl.BlockSpec((D, LM_BN), lambda n: (0, n)),
            pl.BlockSpec((1, LM_BN), lambda n: (0, n)),
        ],
        out_specs=pl.BlockSpec((S, LM_BN), lambda n: (0, n)),
        out_shape=jax.ShapeDtypeStruct((S, V), _F32),
        compiler_params=_ARB(1),
    )(x, w, b)


def kernel(x, emb, Wq, Wkv, Wku, Wvu, Wo, ln1g, ln1b, ln2g, ln2b,
           Wf1, Wf2, cent, We1, We2, Wlm, blm):
    tokens = x.reshape(S).astype(jnp.int32)
    xs = _embed(tokens, emb)
    pos = jnp.arange(S)

    moe_logits = None
    moe_topk = None
    for i in range(L):
        h = _ln(xs[None], ln1g[i], ln1b[i])[0]
        q = _mm(h, Wq[i])
        lat = _mm(h, Wkv[i])
        k = _mm(lat, Wku[i])
        v = _mm(lat, Wvu[i])
        qh = _rope(q.reshape(1, S, H, HD).transpose(0, 2, 1, 3), pos)[0]
        kh = _rope(k.reshape(1, S, H, HD).transpose(0, 2, 1, 3), pos)[0]
        vh = v.reshape(S, H, HD).transpose(1, 0, 2)
        s = _qk(qh, kh)[None]
        m = jnp.max(s, axis=-1, keepdims=True)
        e = jnp.exp(s - m)
        den = jnp.sum(e, axis=-1, keepdims=True)
        o4 = _ev(e[0], vh)[None] / den
        o = o4[0].transpose(1, 0, 2).reshape(S, D)
        xs = _outproj(o, Wo[i], xs)
        h2 = _ln(xs[None], ln2g[i], ln2b[i])[0]
        if i < ND:
            xs = _ffn(h2, xs, Wf1[i], Wf2[i])
        else:
            xs, rl, tk = _moe(h2, xs, cent, We1, We2)
            moe_logits = rl
            moe_topk = tk

    logits = _lm_head(xs, Wlm, blm.reshape(1, V))
    return (logits.reshape(B, S, V),
            moe_logits.reshape(1, B, S, E),
            moe_topk.reshape(1, B, S, K))


# speedup vs baseline: 1.3550x; 1.3550x over previous
import functools

import jax
import jax.numpy as jnp
from jax.experimental import pallas as pl
from jax.experimental.pallas import tpu as pltpu

V = 100000
D = 1024
LAT = 576
H = 16
HD = D // H
L = 4
ND = 3
E = 8
K = 2
FH = 1024
B = 1
S = 2048

EMB_BT = 16
BS = 256
BQ = 256
LM_BN = 1024

_F32 = jnp.float32
_ARB = lambda n: pltpu.CompilerParams(dimension_semantics=("arbitrary",) * n)


def _ln(x, g, b):
    m = jnp.mean(x, axis=-1, keepdims=True)
    v = jnp.var(x, axis=-1, keepdims=True)
    return (x - m) / jnp.sqrt(v + 1e-5) * g + b


def _rope(x, pos):
    half = x.shape[-1] // 2
    freqs = 1.0 / (10000.0 ** (jnp.arange(half, dtype=jnp.float32) / half))
    ang = pos[:, None].astype(jnp.float32) * freqs[None, :]
    cos = jnp.cos(ang)[None, None, :, :]
    sin = jnp.sin(ang)[None, None, :, :]
    x1 = x[..., :half]
    x2 = x[..., half:]
    return jnp.concatenate([x1 * cos - x2 * sin, x1 * sin + x2 * cos], axis=-1)


def _embed_body(ids_ref, *refs):
    del ids_ref
    out_ref = refs[-1]
    for j in range(EMB_BT):
        out_ref[0, pl.ds(j, 1), :] = refs[j][0]


def _embed(tokens, emb):
    def row_map(i, ids, j=0):
        return (ids[i * EMB_BT + j], 0, 0)

    emb3 = emb.reshape(V, 1, D)
    in_specs = [
        pl.BlockSpec((1, 1, D), functools.partial(row_map, j=j))
        for j in range(EMB_BT)
    ]
    grid_spec = pltpu.PrefetchScalarGridSpec(
        num_scalar_prefetch=1,
        grid=(S // EMB_BT,),
        in_specs=in_specs,
        out_specs=pl.BlockSpec((1, EMB_BT, D), lambda i, ids: (i, 0, 0)),
    )
    out = pl.pallas_call(
        _embed_body,
        grid_spec=grid_spec,
        out_shape=jax.ShapeDtypeStruct((S // EMB_BT, EMB_BT, D), _F32),
    )(tokens, *([emb3] * EMB_BT))
    return out.reshape(S, D)


def _mm_body(a_ref, w_ref, o_ref):
    o_ref[...] = jnp.dot(a_ref[...], w_ref[...], preferred_element_type=_F32)


def _mm(a, w):
    ka, nb = w.shape
    return pl.pallas_call(
        _mm_body,
        grid=(S // BS,),
        in_specs=[pl.BlockSpec((BS, ka), lambda i: (i, 0)),
                  pl.BlockSpec((ka, nb), lambda i: (0, 0))],
        out_specs=pl.BlockSpec((BS, nb), lambda i: (i, 0)),
        out_shape=jax.ShapeDtypeStruct((S, nb), _F32),
        compiler_params=_ARB(1),
    )(a, w)


def _attn_body(q_ref, k_ref, v_ref, o_ref):
    qb = pl.program_id(1)
    s = jnp.dot(q_ref[0], k_ref[0].T, preferred_element_type=_F32)
    s = s / jnp.sqrt(jnp.float32(HD))
    row = qb * BQ + jax.lax.broadcasted_iota(jnp.int32, (BQ, S), 0)
    col = jax.lax.broadcasted_iota(jnp.int32, (BQ, S), 1)
    s = jnp.where(col <= row, s, jnp.float32(-1e9))
    m = jnp.max(s, axis=1, keepdims=True)
    e = jnp.exp(s - m)
    acc = jnp.dot(e, v_ref[0], preferred_element_type=_F32)
    o_ref[0] = acc / jnp.sum(e, axis=1, keepdims=True)


def _attn(q, k, v):
    return pl.pallas_call(
        _attn_body,
        grid=(H, S // BQ),
        in_specs=[
            pl.BlockSpec((1, BQ, HD), lambda h, i: (h, i, 0)),
            pl.BlockSpec((1, S, HD), lambda h, i: (h, 0, 0)),
            pl.BlockSpec((1, S, HD), lambda h, i: (h, 0, 0)),
        ],
        out_specs=pl.BlockSpec((1, BQ, HD), lambda h, i: (h, i, 0)),
        out_shape=jax.ShapeDtypeStruct((H, S, HD), _F32),
        compiler_params=pltpu.CompilerParams(
            dimension_semantics=("arbitrary", "arbitrary")),
    )(q, k, v)


def _outproj_body(o_ref, wo_ref, x_ref, out_ref):
    out_ref[...] = x_ref[...] + jnp.dot(
        o_ref[...], wo_ref[...], preferred_element_type=_F32)


def _outproj(o, wo, x):
    row = pl.BlockSpec((BS, D), lambda i: (i, 0))
    return pl.pallas_call(
        _outproj_body,
        grid=(S // BS,),
        in_specs=[row, pl.BlockSpec((D, D), lambda i: (0, 0)), row],
        out_specs=row,
        out_shape=jax.ShapeDtypeStruct((S, D), _F32),
        compiler_params=_ARB(1),
    )(o, wo, x)


def _ffn_body(h_ref, x_ref, w1_ref, w2_ref, out_ref):
    f = jnp.maximum(
        jnp.dot(h_ref[...], w1_ref[...], preferred_element_type=_F32), 0.0)
    out_ref[...] = x_ref[...] + jnp.dot(
        f, w2_ref[...], preferred_element_type=_F32)


def _ffn(h2, x, w1, w2):
    row = pl.BlockSpec((BS, D), lambda i: (i, 0))
    return pl.pallas_call(
        _ffn_body,
        grid=(S // BS,),
        in_specs=[row, row,
                  pl.BlockSpec((D, FH), lambda i: (0, 0)),
                  pl.BlockSpec((FH, D), lambda i: (0, 0))],
        out_specs=row,
        out_shape=jax.ShapeDtypeStruct((S, D), _F32),
        compiler_params=_ARB(1),
    )(h2, x, w1, w2)


def _moe_body(h_ref, x_ref, cent_ref, we1_ref, we2_ref,
              out_ref, rl_ref, tk_ref, acc_ref):
    e = pl.program_id(1)
    h2 = h_ref[...]
    rl = jnp.dot(h2, cent_ref[...].T, preferred_element_type=_F32)

    ii = jax.lax.broadcasted_iota(jnp.int32, (BS, E), 1)
    t1v = jnp.max(rl, axis=1, keepdims=True)
    t1i = jnp.min(jnp.where(rl == t1v, ii, E), axis=1, keepdims=True)
    masked = jnp.where(ii == t1i, -jnp.inf, rl)
    t2v = jnp.max(masked, axis=1, keepdims=True)
    t2i = jnp.min(jnp.where(masked == t2v, ii, E), axis=1, keepdims=True)

    w2 = jnp.exp(t2v - t1v)
    den = 1.0 + w2
    g1 = 1.0 / den
    g2 = w2 / den
    gate = (jnp.where(t1i == e, g1, 0.0) + jnp.where(t2i == e, g2, 0.0))

    he = jnp.maximum(
        jnp.dot(h2, we1_ref[0], preferred_element_type=_F32), 0.0)
    ye = jnp.dot(he, we2_ref[0], preferred_element_type=_F32)
    contrib = ye * gate

    @pl.when(e == 0)
    def _():
        acc_ref[...] = contrib
        rl_ref[...] = rl
        ki = jax.lax.broadcasted_iota(jnp.int32, (BS, K), 1)
        tk_ref[...] = jnp.where(ki == 0, t1i, t2i)

    @pl.when(e > 0)
    def _():
        acc_ref[...] = acc_ref[...] + contrib

    @pl.when(e == E - 1)
    def _():
        out_ref[...] = x_ref[...] + acc_ref[...]


def _moe(h2, x, cent, we1, we2):
    row = pl.BlockSpec((BS, D), lambda i, e: (i, 0))
    return pl.pallas_call(
        _moe_body,
        grid=(S // BS, E),
        in_specs=[
            row,
            row,
            pl.BlockSpec((E, D), lambda i, e: (0, 0)),
            pl.BlockSpec((1, D, FH), lambda i, e: (e, 0, 0)),
            pl.BlockSpec((1, FH, D), lambda i, e: (e, 0, 0)),
        ],
        out_specs=[
            row,
            pl.BlockSpec((BS, E), lambda i, e: (i, 0)),
            pl.BlockSpec((BS, K), lambda i, e: (i, 0)),
        ],
        out_shape=[
            jax.ShapeDtypeStruct((S, D), _F32),
            jax.ShapeDtypeStruct((S, E), _F32),
            jax.ShapeDtypeStruct((S, K), jnp.int32),
        ],
        scratch_shapes=[pltpu.VMEM((BS, D), _F32)],
        compiler_params=pltpu.CompilerParams(
            dimension_semantics=("arbitrary", "arbitrary")),
    )(h2, x, cent, we1, we2)


def _lm_body(x_ref, w_ref, b_ref, out_ref):
    out_ref[...] = jnp.dot(
        x_ref[...], w_ref[...], preferred_element_type=_F32) + b_ref[...]


def _lm_head(x, w, b):
    nb = pl.cdiv(V, LM_BN)
    return pl.pallas_call(
        _lm_body,
        grid=(nb,),
        in_specs=[
            pl.BlockSpec((S, D), lambda n: (0, 0)),
            pl.BlockSpec((D, LM_BN), lambda n: (0, n)),
            pl.BlockSpec((1, LM_BN), lambda n: (0, n)),
        ],
        out_specs=pl.BlockSpec((S, LM_BN), lambda n: (0, n)),
        out_shape=jax.ShapeDtypeStruct((S, V), _F32),
        compiler_params=_ARB(1),
    )(x, w, b)


def kernel(x, emb, Wq, Wkv, Wku, Wvu, Wo, ln1g, ln1b, ln2g, ln2b,
           Wf1, Wf2, cent, We1, We2, Wlm, blm):
    tokens = x.reshape(S).astype(jnp.int32)
    xs = _embed(tokens, emb)
    pos = jnp.arange(S)

    moe_logits = None
    moe_topk = None
    for i in range(L):
        h = _ln(xs[None], ln1g[i], ln1b[i])[0]
        q = _mm(h, Wq[i])
        lat = _mm(h, Wkv[i])
        k = _mm(lat, Wku[i])
        v = _mm(lat, Wvu[i])
        qh = _rope(q.reshape(1, S, H, HD).transpose(0, 2, 1, 3), pos)[0]
        kh = _rope(k.reshape(1, S, H, HD).transpose(0, 2, 1, 3), pos)[0]
        vh = v.reshape(S, H, HD).transpose(1, 0, 2)
        o = _attn(qh, kh, vh).transpose(1, 0, 2).reshape(S, D)
        xs = _outproj(o, Wo[i], xs)
        h2 = _ln(xs[None], ln2g[i], ln2b[i])[0]
        if i < ND:
            xs = _ffn(h2, xs, Wf1[i], Wf2[i])
        else:
            xs, rl, tk = _moe(h2, xs, cent, We1, We2)
            moe_logits = rl
            moe_topk = tk

    logits = _lm_head(xs, Wlm, blm.reshape(1, V))
    return (logits.reshape(B, S, V),
            moe_logits.reshape(1, B, S, E),
            moe_topk.reshape(1, B, S, K))


# speedup vs baseline: 1.3821x; 1.0199x over previous
import functools

import jax
import jax.numpy as jnp
from jax.experimental import pallas as pl
from jax.experimental.pallas import tpu as pltpu

V = 100000
D = 1024
LAT = 576
H = 16
HD = D // H
L = 4
ND = 3
E = 8
K = 2
FH = 1024
B = 1
S = 2048

EMB_BT = 16
BS = 256
BQ = 256
LM_BN = 1024

_F32 = jnp.float32
_ARB = lambda n: pltpu.CompilerParams(dimension_semantics=("arbitrary",) * n)


def _ln(x, g, b):
    m = jnp.mean(x, axis=-1, keepdims=True)
    v = jnp.var(x, axis=-1, keepdims=True)
    return (x - m) / jnp.sqrt(v + 1e-5) * g + b


def _rope(x, pos):
    half = x.shape[-1] // 2
    freqs = 1.0 / (10000.0 ** (jnp.arange(half, dtype=jnp.float32) / half))
    ang = pos[:, None].astype(jnp.float32) * freqs[None, :]
    cos = jnp.cos(ang)[None, None, :, :]
    sin = jnp.sin(ang)[None, None, :, :]
    x1 = x[..., :half]
    x2 = x[..., half:]
    return jnp.concatenate([x1 * cos - x2 * sin, x1 * sin + x2 * cos], axis=-1)


def _embed_body(ids_ref, *refs):
    del ids_ref
    out_ref = refs[-1]
    for j in range(EMB_BT):
        out_ref[0, pl.ds(j, 1), :] = refs[j][0]


def _embed(tokens, emb):
    def row_map(i, ids, j=0):
        return (ids[i * EMB_BT + j], 0, 0)

    emb3 = emb.reshape(V, 1, D)
    in_specs = [
        pl.BlockSpec((1, 1, D), functools.partial(row_map, j=j))
        for j in range(EMB_BT)
    ]
    grid_spec = pltpu.PrefetchScalarGridSpec(
        num_scalar_prefetch=1,
        grid=(S // EMB_BT,),
        in_specs=in_specs,
        out_specs=pl.BlockSpec((1, EMB_BT, D), lambda i, ids: (i, 0, 0)),
    )
    out = pl.pallas_call(
        _embed_body,
        grid_spec=grid_spec,
        out_shape=jax.ShapeDtypeStruct((S // EMB_BT, EMB_BT, D), _F32),
    )(tokens, *([emb3] * EMB_BT))
    return out.reshape(S, D)


def _mm_body(a_ref, w_ref, o_ref):
    o_ref[...] = jnp.dot(a_ref[...], w_ref[...], preferred_element_type=_F32)


def _mm(a, w):
    ka, nb = w.shape
    return pl.pallas_call(
        _mm_body,
        grid=(S // BS,),
        in_specs=[pl.BlockSpec((BS, ka), lambda i: (i, 0)),
                  pl.BlockSpec((ka, nb), lambda i: (0, 0))],
        out_specs=pl.BlockSpec((BS, nb), lambda i: (i, 0)),
        out_shape=jax.ShapeDtypeStruct((S, nb), _F32),
        compiler_params=_ARB(1),
    )(a, w)


def _proj_body(h_ref, wq_ref, wkv_ref, wku_ref, wvu_ref,
               q_ref, k_ref, v_ref):
    h = h_ref[...]
    q_ref[...] = jnp.dot(h, wq_ref[...], preferred_element_type=_F32)
    lat = jnp.dot(h, wkv_ref[...], preferred_element_type=_F32)
    k_ref[...] = jnp.dot(lat, wku_ref[...], preferred_element_type=_F32)
    v_ref[...] = jnp.dot(lat, wvu_ref[...], preferred_element_type=_F32)


def _proj(h, wq, wkv, wku, wvu):
    row = pl.BlockSpec((BS, D), lambda i: (i, 0))
    full = lambda r, c: pl.BlockSpec((r, c), lambda i: (0, 0))
    return pl.pallas_call(
        _proj_body,
        grid=(S // BS,),
        in_specs=[row, full(D, D), full(D, LAT), full(LAT, D), full(LAT, D)],
        out_specs=[row, row, row],
        out_shape=[jax.ShapeDtypeStruct((S, D), _F32)] * 3,
        compiler_params=_ARB(1),
    )(h, wq, wkv, wku, wvu)


def _attn_body(q_ref, k_ref, v_ref, o_ref):
    qb = pl.program_id(1)
    s = jnp.dot(q_ref[0], k_ref[0].T, preferred_element_type=_F32)
    s = s / jnp.sqrt(jnp.float32(HD))
    row = qb * BQ + jax.lax.broadcasted_iota(jnp.int32, (BQ, S), 0)
    col = jax.lax.broadcasted_iota(jnp.int32, (BQ, S), 1)
    s = jnp.where(col <= row, s, jnp.float32(-1e9))
    m = jnp.max(s, axis=1, keepdims=True)
    e = jnp.exp(s - m)
    acc = jnp.dot(e, v_ref[0], preferred_element_type=_F32)
    o_ref[0] = acc / jnp.sum(e, axis=1, keepdims=True)


def _attn(q, k, v):
    return pl.pallas_call(
        _attn_body,
        grid=(H, S // BQ),
        in_specs=[
            pl.BlockSpec((1, BQ, HD), lambda h, i: (h, i, 0)),
            pl.BlockSpec((1, S, HD), lambda h, i: (h, 0, 0)),
            pl.BlockSpec((1, S, HD), lambda h, i: (h, 0, 0)),
        ],
        out_specs=pl.BlockSpec((1, BQ, HD), lambda h, i: (h, i, 0)),
        out_shape=jax.ShapeDtypeStruct((H, S, HD), _F32),
        compiler_params=pltpu.CompilerParams(
            dimension_semantics=("arbitrary", "arbitrary")),
    )(q, k, v)


def _outproj_body(o_ref, wo_ref, x_ref, out_ref):
    out_ref[...] = x_ref[...] + jnp.dot(
        o_ref[...], wo_ref[...], preferred_element_type=_F32)


def _outproj(o, wo, x):
    row = pl.BlockSpec((BS, D), lambda i: (i, 0))
    return pl.pallas_call(
        _outproj_body,
        grid=(S // BS,),
        in_specs=[row, pl.BlockSpec((D, D), lambda i: (0, 0)), row],
        out_specs=row,
        out_shape=jax.ShapeDtypeStruct((S, D), _F32),
        compiler_params=_ARB(1),
    )(o, wo, x)


def _ffn_body(h_ref, x_ref, w1_ref, w2_ref, out_ref):
    f = jnp.maximum(
        jnp.dot(h_ref[...], w1_ref[...], preferred_element_type=_F32), 0.0)
    out_ref[...] = x_ref[...] + jnp.dot(
        f, w2_ref[...], preferred_element_type=_F32)


def _ffn(h2, x, w1, w2):
    row = pl.BlockSpec((BS, D), lambda i: (i, 0))
    return pl.pallas_call(
        _ffn_body,
        grid=(S // BS,),
        in_specs=[row, row,
                  pl.BlockSpec((D, FH), lambda i: (0, 0)),
                  pl.BlockSpec((FH, D), lambda i: (0, 0))],
        out_specs=row,
        out_shape=jax.ShapeDtypeStruct((S, D), _F32),
        compiler_params=_ARB(1),
    )(h2, x, w1, w2)


def _moe_body(h_ref, x_ref, cent_ref, we1_ref, we2_ref,
              out_ref, rl_ref, tk_ref, acc_ref):
    e = pl.program_id(1)
    h2 = h_ref[...]
    rl = jnp.dot(h2, cent_ref[...].T, preferred_element_type=_F32)

    ii = jax.lax.broadcasted_iota(jnp.int32, (BS, E), 1)
    t1v = jnp.max(rl, axis=1, keepdims=True)
    t1i = jnp.min(jnp.where(rl == t1v, ii, E), axis=1, keepdims=True)
    masked = jnp.where(ii == t1i, -jnp.inf, rl)
    t2v = jnp.max(masked, axis=1, keepdims=True)
    t2i = jnp.min(jnp.where(masked == t2v, ii, E), axis=1, keepdims=True)

    w2 = jnp.exp(t2v - t1v)
    den = 1.0 + w2
    g1 = 1.0 / den
    g2 = w2 / den
    gate = (jnp.where(t1i == e, g1, 0.0) + jnp.where(t2i == e, g2, 0.0))

    he = jnp.maximum(
        jnp.dot(h2, we1_ref[0], preferred_element_type=_F32), 0.0)
    ye = jnp.dot(he, we2_ref[0], preferred_element_type=_F32)
    contrib = ye * gate

    @pl.when(e == 0)
    def _():
        acc_ref[...] = contrib
        rl_ref[...] = rl
        ki = jax.lax.broadcasted_iota(jnp.int32, (BS, K), 1)
        tk_ref[...] = jnp.where(ki == 0, t1i, t2i)

    @pl.when(e > 0)
    def _():
        acc_ref[...] = acc_ref[...] + contrib

    @pl.when(e == E - 1)
    def _():
        out_ref[...] = x_ref[...] + acc_ref[...]


def _moe(h2, x, cent, we1, we2):
    row = pl.BlockSpec((BS, D), lambda i, e: (i, 0))
    return pl.pallas_call(
        _moe_body,
        grid=(S // BS, E),
        in_specs=[
            row,
            row,
            pl.BlockSpec((E, D), lambda i, e: (0, 0)),
            pl.BlockSpec((1, D, FH), lambda i, e: (e, 0, 0)),
            pl.BlockSpec((1, FH, D), lambda i, e: (e, 0, 0)),
        ],
        out_specs=[
            row,
            pl.BlockSpec((BS, E), lambda i, e: (i, 0)),
            pl.BlockSpec((BS, K), lambda i, e: (i, 0)),
        ],
        out_shape=[
            jax.ShapeDtypeStruct((S, D), _F32),
            jax.ShapeDtypeStruct((S, E), _F32),
            jax.ShapeDtypeStruct((S, K), jnp.int32),
        ],
        scratch_shapes=[pltpu.VMEM((BS, D), _F32)],
        compiler_params=pltpu.CompilerParams(
            dimension_semantics=("arbitrary", "arbitrary")),
    )(h2, x, cent, we1, we2)


def _lm_body(x_ref, w_ref, b_ref, out_ref):
    out_ref[...] = jnp.dot(
        x_ref[...], w_ref[...], preferred_element_type=_F32) + b_ref[...]


def _lm_head(x, w, b):
    nb = pl.cdiv(V, LM_BN)
    return pl.pallas_call(
        _lm_body,
        grid=(nb,),
        in_specs=[
            pl.BlockSpec((S, D), lambda n: (0, 0)),
            pl.BlockSpec((D, LM_BN), lambda n: (0, n)),
            pl.BlockSpec((1, LM_BN), lambda n: (0, n)),
        ],
        out_specs=pl.BlockSpec((S, LM_BN), lambda n: (0, n)),
        out_shape=jax.ShapeDtypeStruct((S, V), _F32),
        compiler_params=_ARB(1),
    )(x, w, b)


def kernel(x, emb, Wq, Wkv, Wku, Wvu, Wo, ln1g, ln1b, ln2g, ln2b,
           Wf1, Wf2, cent, We1, We2, Wlm, blm):
    tokens = x.reshape(S).astype(jnp.int32)
    xs = _embed(tokens, emb)
    pos = jnp.arange(S)

    moe_logits = None
    moe_topk = None
    for i in range(L):
        h = _ln(xs[None], ln1g[i], ln1b[i])[0]
        q, k, v = _proj(h, Wq[i], Wkv[i], Wku[i], Wvu[i])
        qh = _rope(q.reshape(1, S, H, HD).transpose(0, 2, 1, 3), pos)[0]
        kh = _rope(k.reshape(1, S, H, HD).transpose(0, 2, 1, 3), pos)[0]
        vh = v.reshape(S, H, HD).transpose(1, 0, 2)
        o = _attn(qh, kh, vh).transpose(1, 0, 2).reshape(S, D)
        xs = _outproj(o, Wo[i], xs)
        h2 = _ln(xs[None], ln2g[i], ln2b[i])[0]
        if i < ND:
            xs = _ffn(h2, xs, Wf1[i], Wf2[i])
        else:
            xs, rl, tk = _moe(h2, xs, cent, We1, We2)
            moe_logits = rl
            moe_topk = tk

    logits = _lm_head(xs, Wlm, blm.reshape(1, V))
    return (logits.reshape(B, S, V),
            moe_logits.reshape(1, B, S, E),
            moe_topk.reshape(1, B, S, K))
